# baseline (device time: 86384 ns/iter reference)
import jax
import jax.numpy as jnp
from jax import lax
from jax.experimental import pallas as pl
from jax.experimental.pallas import tpu as pltpu

N_DEV = 4
B, SQ, SKV = 2, 512, 512
HQ_SHARD, DH = 8, 64
DM = 768
HD = HQ_SHARD * DH


def kernel(x, Wq, K_ext, V_ext, Wo):
    def body(x_ref, wq_ref, k_ref, v_ref, wo_ref, out_ref,
             ctx_ref, comm_ref, send_sems, recv_sems):
        my = lax.axis_index("i")
        left = lax.rem(my + N_DEV - 1, N_DEV)
        right = lax.rem(my + 1, N_DEV)

        barrier_sem = pltpu.get_barrier_semaphore()
        for nbr in (left, right):
            pl.semaphore_signal(
                barrier_sem, inc=1,
                device_id=(nbr,), device_id_type=pl.DeviceIdType.MESH,
            )
        pl.semaphore_wait(barrier_sem, 2)

        wq_slice = wq_ref[:, pl.ds(my * HD, HD)].astype(jnp.bfloat16)
        x2 = x_ref[...].reshape(B * SQ, DM).astype(jnp.bfloat16)
        q = (lax.dot(x2, wq_slice, preferred_element_type=jnp.float32)
             * 0.125).astype(jnp.bfloat16)

        ri = lax.broadcasted_iota(jnp.int32, (SQ, SKV), 0)
        ci = lax.broadcasted_iota(jnp.int32, (SQ, SKV), 1)
        mask = ((ri // 64) % 4) == ((ci // 64) % 4)

        for b in range(B):
            for h in range(HQ_SHARD):
                q_bh = q[b * SQ:(b + 1) * SQ, h * DH:(h + 1) * DH]
                k_bh = k_ref[b, :, h, :].astype(jnp.bfloat16)
                s = lax.dot_general(
                    q_bh, k_bh, (((1,), (1,)), ((), ())),
                    preferred_element_type=jnp.float32)
                s = jnp.where(mask, s, -1e9)
                m = jnp.max(s, axis=1, keepdims=True)
                w = jnp.exp(s - m)
                w = (w / jnp.sum(w, axis=1, keepdims=True)).astype(jnp.bfloat16)
                v_bh = v_ref[b, :, h, :].astype(jnp.bfloat16)
                c = lax.dot(w, v_bh, preferred_element_type=jnp.float32)
                ctx_ref[b, :, pl.ds(h * DH, DH)] = c.astype(jnp.bfloat16)

        wo_slice = wo_ref[pl.ds(my * HD, HD), :].astype(jnp.bfloat16)
        for b in range(B):
            pb = lax.dot(ctx_ref[b], wo_slice,
                         preferred_element_type=jnp.float32)
            out_ref[b] = pb
            comm_ref[0, pl.ds(b * SQ, SQ), :] = pb.astype(jnp.bfloat16)

        for hop in range(N_DEV - 1):
            rdma = pltpu.make_async_remote_copy(
                src_ref=comm_ref.at[hop],
                dst_ref=comm_ref.at[hop + 1],
                send_sem=send_sems.at[hop],
                recv_sem=recv_sems.at[hop],
                device_id=(right,),
                device_id_type=pl.DeviceIdType.MESH,
            )
            rdma.start()
            rdma.wait()
            inc = comm_ref[hop + 1].reshape(B, SQ, DM).astype(jnp.float32)
            out_ref[...] = out_ref[...] + inc

    return pl.pallas_call(
        body,
        out_shape=jax.ShapeDtypeStruct((B, SQ, DM), jnp.float32),
        in_specs=[pl.BlockSpec(memory_space=pltpu.VMEM)] * 5,
        out_specs=pl.BlockSpec(memory_space=pltpu.VMEM),
        scratch_shapes=[
            pltpu.VMEM((B, SQ, HD), jnp.bfloat16),
            pltpu.VMEM((N_DEV, B * SQ, DM), jnp.bfloat16),
            pltpu.SemaphoreType.DMA((N_DEV - 1,)),
            pltpu.SemaphoreType.DMA((N_DEV - 1,)),
        ],
        compiler_params=pltpu.CompilerParams(collective_id=0),
    )(x, Wq, K_ext, V_ext, Wo)


# device time: 36884 ns/iter; 2.3420x vs baseline; 2.3420x over previous
import jax
import jax.numpy as jnp
from jax import lax
from jax.experimental import pallas as pl
from jax.experimental.pallas import tpu as pltpu

N_DEV = 4
B, SQ, SKV = 2, 512, 512
HQ_SHARD, DH = 8, 64
DM = 768
HD = HQ_SHARD * DH
SUB = 128
BLK = 64


def kernel(x, Wq, K_ext, V_ext, Wo):
    def body(x_ref, wq_ref, k_ref, v_ref, wo_ref, out_ref,
             wq_s, wo_s, qh_ref, ctxb_ref,
             pb_ref, rs_buf, ag_src, ag_buf,
             wq_sem, wo_sem,
             rs_send_sems, rs_recv_sems, ag_send_sems, ag_recv_sems):
        my = lax.axis_index("i")
        right = lax.rem(my + 1, N_DEV)
        opp = lax.rem(my + 2, N_DEV)
        left = lax.rem(my + 3, N_DEV)
        peers = (right, opp, left)

        def dir_slot(dest):
            return jnp.where(dest == right, 0, jnp.where(dest == opp, 1, 2))

        wq_cp = pltpu.make_async_copy(
            wq_ref.at[:, pl.ds(my * HD, HD)], wq_s, wq_sem)
        wq_cp.start()
        wo_cp = pltpu.make_async_copy(
            wo_ref.at[pl.ds(my * HD, HD), :], wo_s, wo_sem)
        wo_cp.start()

        barrier_sem = pltpu.get_barrier_semaphore()
        for nbr in peers:
            pl.semaphore_signal(
                barrier_sem, inc=1,
                device_id=(nbr,), device_id_type=pl.DeviceIdType.MESH,
            )
        pl.semaphore_wait(barrier_sem, 3)

        wq_cp.wait()
        x2 = x_ref[...].reshape(B * SQ, DM)
        q = (lax.dot(x2, wq_s[...], preferred_element_type=jnp.float32)
             * 0.125).astype(jnp.bfloat16)
        for h in range(HQ_SHARD):
            qh_ref[h] = q[:, h * DH:(h + 1) * DH]
        wo_cp.wait()
        wo_slice = wo_s[...]

        def quarter(b, half):
            for r in (2 * half, 2 * half + 1):
                r0, r1 = BLK * r, 4 * BLK + BLK * r
                qg = jnp.concatenate([
                    qh_ref[:, b * SQ + r0:b * SQ + r0 + BLK, :],
                    qh_ref[:, b * SQ + r1:b * SQ + r1 + BLK, :],
                ], axis=1)
                kg = jnp.concatenate([
                    k_ref[b, :, r0:r0 + BLK, :],
                    k_ref[b, :, r1:r1 + BLK, :],
                ], axis=1)
                s = lax.dot_general(
                    qg, kg, (((2,), (2,)), ((0,), (0,))),
                    preferred_element_type=jnp.float32)
                w = jnp.exp(s)
                w = (w / jnp.sum(w, axis=2, keepdims=True)).astype(jnp.bfloat16)
                vg = jnp.concatenate([
                    v_ref[b, :, r0:r0 + BLK, :],
                    v_ref[b, :, r1:r1 + BLK, :],
                ], axis=1)
                cg = lax.dot_general(
                    w, vg, (((2,), (1,)), ((0,), (0,))),
                    preferred_element_type=jnp.float32
                ).astype(jnp.bfloat16)
                ctxb_ref[:, r0:r0 + BLK, :] = cg[:, :BLK, :]
                ctxb_ref[:, r1:r1 + BLK, :] = cg[:, BLK:, :]
            jl, jh = half, half + 2
            lo, hi = SUB * jl, SUB * jh
            ctx_l = jnp.concatenate(
                [ctxb_ref[h, lo:lo + SUB, :] for h in range(HQ_SHARD)], axis=1)
            ctx_h = jnp.concatenate(
                [ctxb_ref[h, hi:hi + SUB, :] for h in range(HQ_SHARD)], axis=1)
            p = lax.dot(jnp.concatenate([ctx_l, ctx_h], axis=0), wo_slice,
                        preferred_element_type=jnp.float32
                        ).astype(jnp.bfloat16)
            pb_ref[b, lo:lo + SUB, :] = p[:SUB]
            pb_ref[b, hi:hi + SUB, :] = p[SUB:]
            sends = []
            for j in (jl, jh):
                d = pltpu.make_async_remote_copy(
                    src_ref=pb_ref.at[b, pl.ds(j * SUB, SUB), :],
                    dst_ref=rs_buf.at[b, dir_slot(j)],
                    send_sem=rs_send_sems.at[N_DEV * b + j],
                    recv_sem=rs_recv_sems.at[3 * b + dir_slot(j)],
                    device_id=(j,),
                    device_id_type=pl.DeviceIdType.MESH,
                )

                @pl.when(my != j)
                def _():
                    d.start()

                sends.append((j, d))
            return sends

        def reduce_and_ag(b):
            red = pb_ref[b, pl.ds(my * SUB, SUB), :].astype(jnp.float32)
            for p in range(N_DEV - 1):
                d = pltpu.make_async_remote_copy(
                    src_ref=pb_ref.at[b, pl.ds(0, SUB), :],
                    dst_ref=rs_buf.at[b, p],
                    send_sem=rs_send_sems.at[0],
                    recv_sem=rs_recv_sems.at[3 * b + p],
                    device_id=(my,),
                    device_id_type=pl.DeviceIdType.MESH,
                )
                d.wait_recv()
                red = red + rs_buf[b, p].astype(jnp.float32)
            redb = red.astype(jnp.bfloat16)
            out_ref[pl.ds(b * SQ + my * SUB, SUB), :] = redb
            ag_src[b] = redb
            ags = []
            for p, dest in enumerate(peers):
                d = pltpu.make_async_remote_copy(
                    src_ref=ag_src.at[b],
                    dst_ref=ag_buf.at[b, dir_slot(dest)],
                    send_sem=ag_send_sems.at[3 * b + p],
                    recv_sem=ag_recv_sems.at[3 * b + dir_slot(dest)],
                    device_id=(dest,),
                    device_id_type=pl.DeviceIdType.MESH,
                )
                d.start()
                ags.append(d)
            return ags

        def ag_wait(b):
            for p, src_chunk in enumerate((left, opp, right)):
                d = pltpu.make_async_remote_copy(
                    src_ref=ag_src.at[b],
                    dst_ref=ag_buf.at[b, p],
                    send_sem=ag_send_sems.at[0],
                    recv_sem=ag_recv_sems.at[3 * b + p],
                    device_id=(my,),
                    device_id_type=pl.DeviceIdType.MESH,
                )
                d.wait_recv()
                out_ref[pl.ds(b * SQ + src_chunk * SUB, SUB), :] = ag_buf[b, p]

        s00 = quarter(0, 0)
        s01 = quarter(0, 1)
        s10 = quarter(1, 0)
        ag0 = reduce_and_ag(0)
        s11 = quarter(1, 1)
        ag1 = reduce_and_ag(1)
        ag_wait(0)
        ag_wait(1)

        for j, d in s00 + s01 + s10 + s11:
            @pl.when(my != j)
            def _():
                d.wait_send()
        for d in ag0 + ag1:
            d.wait_send()

    x = x.astype(jnp.bfloat16)
    Wq = Wq.astype(jnp.bfloat16)
    Wo = Wo.astype(jnp.bfloat16)
    K_ext = jnp.transpose(K_ext.astype(jnp.bfloat16), (0, 2, 1, 3))
    V_ext = jnp.transpose(V_ext.astype(jnp.bfloat16), (0, 2, 1, 3))
    out2 = pl.pallas_call(
        body,
        out_shape=jax.ShapeDtypeStruct((B * SQ, DM), jnp.bfloat16),
        in_specs=[
            pl.BlockSpec(memory_space=pltpu.VMEM),
            pl.BlockSpec(memory_space=pltpu.HBM),
            pl.BlockSpec(memory_space=pltpu.VMEM),
            pl.BlockSpec(memory_space=pltpu.VMEM),
            pl.BlockSpec(memory_space=pltpu.HBM),
        ],
        out_specs=pl.BlockSpec(memory_space=pltpu.VMEM),
        scratch_shapes=[
            pltpu.VMEM((DM, HD), jnp.bfloat16),
            pltpu.VMEM((HD, DM), jnp.bfloat16),
            pltpu.VMEM((HQ_SHARD, B * SQ, DH), jnp.bfloat16),
            pltpu.VMEM((HQ_SHARD, SQ, DH), jnp.bfloat16),
            pltpu.VMEM((B, SQ, DM), jnp.bfloat16),
            pltpu.VMEM((B, 3, SUB, DM), jnp.bfloat16),
            pltpu.VMEM((B, SUB, DM), jnp.bfloat16),
            pltpu.VMEM((B, 3, SUB, DM), jnp.bfloat16),
            pltpu.SemaphoreType.DMA,
            pltpu.SemaphoreType.DMA,
            pltpu.SemaphoreType.DMA((B * N_DEV,)),
            pltpu.SemaphoreType.DMA((B * 3,)),
            pltpu.SemaphoreType.DMA((B * 3,)),
            pltpu.SemaphoreType.DMA((B * 3,)),
        ],
        compiler_params=pltpu.CompilerParams(collective_id=0),
    )(x, Wq, K_ext, V_ext, Wo)
    return out2.reshape(B, SQ, DM)
